# Initial kernel scaffold; baseline (speedup 1.0000x reference)
import jax
import jax.numpy as jnp
from jax.experimental import pallas as pl


def kernel(x, y, indices, W, b):
    raise NotImplementedError("write your pallas kernel here")


# speedup vs baseline: 22.4896x; 22.4896x over previous
import functools

import jax
import jax.numpy as jnp
from jax.experimental import pallas as pl
from jax.experimental.pallas import tpu as pltpu

B, C_IN, C_OUT, N, M, K = 8, 64, 64, 8192, 512, 8
BLK_N = 1024


def _knn_conv_kernel(x_ref, y_ref, z_ref, wt_ref, bias_ref, out_ref,
                     u_ref, yn_ref):
    i = pl.program_id(1)

    @pl.when(i == 0)
    def _prep():
        yv = y_ref[0]
        norm = jnp.sqrt(jnp.sum(yv * yv, axis=0, keepdims=True))
        yn_ref[...] = (yv / jnp.clip(norm, 1e-12, None)).astype(jnp.bfloat16)
        zv = z_ref[0].astype(jnp.bfloat16)
        for k in range(1, K):
            u_ref[k - 1] = jax.lax.dot_general(
                wt_ref[k].astype(jnp.bfloat16), zv, (((1,), (0,)), ((), ())),
                preferred_element_type=jnp.float32)

    xb = x_ref[0]
    xnorm = jnp.sqrt(jnp.sum(xb * xb, axis=0, keepdims=True))
    xn = (xb / jnp.clip(xnorm, 1e-12, None)).astype(jnp.bfloat16)
    sim = jax.lax.dot_general(
        yn_ref[...], xn, (((0,), (0,)), ((), ())),
        preferred_element_type=jnp.float32)
    acc = jax.lax.dot_general(
        wt_ref[0].astype(jnp.bfloat16), xb.astype(jnp.bfloat16),
        (((1,), (0,)), ((), ())),
        preferred_element_type=jnp.float32) + bias_ref[...]

    iota_m = jax.lax.broadcasted_iota(jnp.int32, (M, BLK_N), 0)
    for k in range(K - 1):
        mx = jnp.max(sim, axis=0, keepdims=True)
        sel = sim == mx
        cmin = jnp.min(jnp.where(sel, iota_m, M), axis=0, keepdims=True)
        onehot = iota_m == cmin
        acc = acc + jax.lax.dot_general(
            u_ref[k], onehot.astype(jnp.float32), (((1,), (0,)), ((), ())),
            preferred_element_type=jnp.float32,
            precision=jax.lax.Precision.HIGHEST)
        sim = jnp.where(onehot, -1e30, sim)

    out_ref[0] = acc


@functools.partial(jax.jit, static_argnames=("interpret",))
def kernel(x, y, indices, W, b, interpret=False):
    z = jnp.take(x, indices, axis=2)
    wt = jnp.transpose(W, (2, 0, 1))
    bias = b.reshape(C_OUT, 1)

    grid = (B, N // BLK_N)
    out = pl.pallas_call(
        _knn_conv_kernel,
        grid=grid,
        in_specs=[
            pl.BlockSpec((1, C_IN, BLK_N), lambda bb, ii: (bb, 0, ii)),
            pl.BlockSpec((1, C_IN, M), lambda bb, ii: (bb, 0, 0)),
            pl.BlockSpec((1, C_IN, M), lambda bb, ii: (bb, 0, 0)),
            pl.BlockSpec((K, C_OUT, C_IN), lambda bb, ii: (0, 0, 0)),
            pl.BlockSpec((C_OUT, 1), lambda bb, ii: (0, 0)),
        ],
        out_specs=pl.BlockSpec((1, C_OUT, BLK_N), lambda bb, ii: (bb, 0, ii)),
        out_shape=jax.ShapeDtypeStruct((B, C_OUT, N), jnp.float32),
        scratch_shapes=[
            pltpu.VMEM((K - 1, C_OUT, M), jnp.float32),
            pltpu.VMEM((C_IN, M), jnp.bfloat16),
        ],
        compiler_params=pltpu.CompilerParams(
            dimension_semantics=("parallel", "arbitrary")),
        interpret=interpret,
    )(x, y, z, wt, bias)
    return out


# speedup vs baseline: 42.8716x; 1.9063x over previous
import functools

import jax
import jax.numpy as jnp
from jax.experimental import pallas as pl
from jax.experimental.pallas import tpu as pltpu

B, C_IN, C_OUT, N, M, K = 8, 64, 64, 8192, 512, 8
BLK_N = 1024


def _knn_conv_kernel(x_ref, y_ref, z_ref, wt_ref, bias_ref, out_ref,
                     uhi_ref, ulo_ref, yn_ref):
    i = pl.program_id(1)

    @pl.when(i == 0)
    def _prep():
        yv = y_ref[0]
        norm = jnp.sqrt(jnp.sum(yv * yv, axis=0, keepdims=True))
        yn_ref[...] = (yv / jnp.clip(norm, 1e-12, None)).astype(jnp.bfloat16)
        zv = z_ref[0].astype(jnp.bfloat16)
        for k in range(1, K):
            u = jax.lax.dot_general(
                wt_ref[k].astype(jnp.bfloat16), zv, (((1,), (0,)), ((), ())),
                preferred_element_type=jnp.float32)
            uhi = u.astype(jnp.bfloat16)
            uhi_ref[k - 1] = uhi
            ulo_ref[k - 1] = (u - uhi.astype(jnp.float32)).astype(jnp.bfloat16)

    xb = x_ref[0]
    xnorm = jnp.sqrt(jnp.sum(xb * xb, axis=0, keepdims=True))
    xn = (xb / jnp.clip(xnorm, 1e-12, None)).astype(jnp.bfloat16)
    sim = jax.lax.dot_general(
        yn_ref[...], xn, (((0,), (0,)), ((), ())),
        preferred_element_type=jnp.float32)
    acc = jax.lax.dot_general(
        wt_ref[0].astype(jnp.bfloat16), xb.astype(jnp.bfloat16),
        (((1,), (0,)), ((), ())),
        preferred_element_type=jnp.float32) + bias_ref[...]

    iota_m = jax.lax.broadcasted_iota(jnp.int32, (M, BLK_N), 0)
    for k in range(K - 1):
        mx = jnp.max(sim, axis=0, keepdims=True)
        sel = sim == mx
        cmin = jnp.min(jnp.where(sel, iota_m, M), axis=0, keepdims=True)
        onehot = iota_m == cmin
        oh = onehot.astype(jnp.bfloat16)
        acc = acc + jax.lax.dot_general(
            uhi_ref[k], oh, (((1,), (0,)), ((), ())),
            preferred_element_type=jnp.float32)
        acc = acc + jax.lax.dot_general(
            ulo_ref[k], oh, (((1,), (0,)), ((), ())),
            preferred_element_type=jnp.float32)
        sim = jnp.where(onehot, -1e30, sim)

    out_ref[0] = acc


@functools.partial(jax.jit, static_argnames=("interpret",))
def kernel(x, y, indices, W, b, interpret=False):
    z = jnp.take(x, indices, axis=2)
    wt = jnp.transpose(W, (2, 0, 1))
    bias = b.reshape(C_OUT, 1)

    grid = (B, N // BLK_N)
    out = pl.pallas_call(
        _knn_conv_kernel,
        grid=grid,
        in_specs=[
            pl.BlockSpec((1, C_IN, BLK_N), lambda bb, ii: (bb, 0, ii)),
            pl.BlockSpec((1, C_IN, M), lambda bb, ii: (bb, 0, 0)),
            pl.BlockSpec((1, C_IN, M), lambda bb, ii: (bb, 0, 0)),
            pl.BlockSpec((K, C_OUT, C_IN), lambda bb, ii: (0, 0, 0)),
            pl.BlockSpec((C_OUT, 1), lambda bb, ii: (0, 0)),
        ],
        out_specs=pl.BlockSpec((1, C_OUT, BLK_N), lambda bb, ii: (bb, 0, ii)),
        out_shape=jax.ShapeDtypeStruct((B, C_OUT, N), jnp.float32),
        scratch_shapes=[
            pltpu.VMEM((K - 1, C_OUT, M), jnp.bfloat16),
            pltpu.VMEM((K - 1, C_OUT, M), jnp.bfloat16),
            pltpu.VMEM((C_IN, M), jnp.bfloat16),
        ],
        compiler_params=pltpu.CompilerParams(
            dimension_semantics=("parallel", "arbitrary")),
        interpret=interpret,
    )(x, y, z, wt, bias)
    return out


# speedup vs baseline: 47.5211x; 1.1085x over previous
import functools

import jax
import jax.numpy as jnp
from jax.experimental import pallas as pl
from jax.experimental.pallas import tpu as pltpu

B, C_IN, C_OUT, N, M, K = 8, 64, 64, 8192, 512, 8
BLK_N = 2048


def _knn_conv_kernel(x_ref, y_ref, z_ref, wt_ref, bias_ref, out_ref,
                     uhi_ref, ulo_ref, yn_ref):
    i = pl.program_id(1)

    @pl.when(i == 0)
    def _prep():
        yv = y_ref[0]
        norm = jnp.sqrt(jnp.sum(yv * yv, axis=0, keepdims=True))
        yn_ref[...] = (yv / jnp.clip(norm, 1e-12, None)).astype(jnp.bfloat16)
        zv = z_ref[0].astype(jnp.bfloat16)
        for k in range(1, K):
            u = jax.lax.dot_general(
                wt_ref[k].astype(jnp.bfloat16), zv, (((1,), (0,)), ((), ())),
                preferred_element_type=jnp.float32)
            uhi = u.astype(jnp.bfloat16)
            uhi_ref[k - 1] = uhi
            ulo_ref[k - 1] = (u - uhi.astype(jnp.float32)).astype(jnp.bfloat16)

    xb = x_ref[0]
    xnorm = jnp.sqrt(jnp.sum(xb * xb, axis=0, keepdims=True))
    xn = (xb / jnp.clip(xnorm, 1e-12, None)).astype(jnp.bfloat16)
    sim = jax.lax.dot_general(
        yn_ref[...], xn, (((0,), (0,)), ((), ())),
        preferred_element_type=jnp.float32)
    acc = jax.lax.dot_general(
        wt_ref[0].astype(jnp.bfloat16), xb.astype(jnp.bfloat16),
        (((1,), (0,)), ((), ())),
        preferred_element_type=jnp.float32) + bias_ref[...]

    iota_m = jax.lax.broadcasted_iota(
        jnp.int32, (M, BLK_N), 0).astype(jnp.float32)
    fm = jnp.float32(M)
    for k in range(K - 1):
        mx = jnp.max(sim, axis=0, keepdims=True)
        cmin = jnp.min(jnp.where(sim == mx, iota_m, fm), axis=0, keepdims=True)
        onehot = iota_m == cmin
        oh = onehot.astype(jnp.bfloat16)
        acc = acc + jax.lax.dot_general(
            uhi_ref[k], oh, (((1,), (0,)), ((), ())),
            preferred_element_type=jnp.float32)
        acc = acc + jax.lax.dot_general(
            ulo_ref[k], oh, (((1,), (0,)), ((), ())),
            preferred_element_type=jnp.float32)
        sim = jnp.where(onehot, -1e30, sim)

    out_ref[0] = acc


@functools.partial(jax.jit, static_argnames=("interpret",))
def kernel(x, y, indices, W, b, interpret=False):
    z = jnp.take(x, indices, axis=2)
    wt = jnp.transpose(W, (2, 0, 1))
    bias = b.reshape(C_OUT, 1)

    grid = (B, N // BLK_N)
    out = pl.pallas_call(
        _knn_conv_kernel,
        grid=grid,
        in_specs=[
            pl.BlockSpec((1, C_IN, BLK_N), lambda bb, ii: (bb, 0, ii)),
            pl.BlockSpec((1, C_IN, M), lambda bb, ii: (bb, 0, 0)),
            pl.BlockSpec((1, C_IN, M), lambda bb, ii: (bb, 0, 0)),
            pl.BlockSpec((K, C_OUT, C_IN), lambda bb, ii: (0, 0, 0)),
            pl.BlockSpec((C_OUT, 1), lambda bb, ii: (0, 0)),
        ],
        out_specs=pl.BlockSpec((1, C_OUT, BLK_N), lambda bb, ii: (bb, 0, ii)),
        out_shape=jax.ShapeDtypeStruct((B, C_OUT, N), jnp.float32),
        scratch_shapes=[
            pltpu.VMEM((K - 1, C_OUT, M), jnp.bfloat16),
            pltpu.VMEM((K - 1, C_OUT, M), jnp.bfloat16),
            pltpu.VMEM((C_IN, M), jnp.bfloat16),
        ],
        compiler_params=pltpu.CompilerParams(
            dimension_semantics=("parallel", "arbitrary")),
        interpret=interpret,
    )(x, y, z, wt, bias)
    return out
